# baseline (speedup 1.0000x reference)
import jax
import jax.numpy as jnp
from jax import lax
from jax.experimental import pallas as pl
from jax.experimental.pallas import tpu as pltpu
from jax.experimental.pallas import tpu_sc as plsc

B = 16384
K = 32
N = 1000000
G = 8
NC = 2
NS = 16
NW = NC * NS
BPW = B // NW
L = 16
W = 16
NWAVES = BPW // W

_MESH = dict(core_axis_name="c", subcore_axis_name="s",
             num_cores=NC, num_subcores=NS)
_PARAMS = pltpu.CompilerParams(needs_layout_passes=False,
                               use_tc_tiling_on_sc=True)


def _worker_base():
    return (lax.axis_index("s") * NC + lax.axis_index("c")) * BPW


def _fire(tab, idx_s, stag, sem, w, half):
    b0 = pl.multiple_of(w * W, W)
    hb = pl.multiple_of(half * W * G, W * G)
    iv = idx_s[pl.ds(b0, W)]
    for t in range(W):
        g = iv[t] >> 3
        pltpu.async_copy(tab.at[g], stag.at[pl.ds(hb + t * G, G)], sem)


def _drain_extract(tab, idx_s, stag, flat, sem, w, half):
    b0 = pl.multiple_of(w * W, W)
    hb = pl.multiple_of(half * W * G, W * G)
    iv = idx_s[pl.ds(b0, W)]
    for t in range(W):
        pltpu.make_async_copy(
            tab.at[0], stag.at[pl.ds(hb + t * G, G)], sem).wait()
    for t in range(W):
        b = b0 + t
        r = hb + t * G + (iv[t] & 7)
        flat[pl.ds(b * K, L)] = stag[r, pl.ds(0, L)]
        flat[pl.ds(b * K + L, L)] = stag[r, pl.ds(L, L)]
    return b0


def _gather_body(ut, uid, out_hbm, idx_s, stag, u_flat, sem):
    base = _worker_base()
    pltpu.sync_copy(uid.at[pl.ds(base, BPW)], idx_s)
    _fire(ut, idx_s, stag, sem, 0, 0)

    def wave(w, _):
        _fire(ut, idx_s, stag, sem, w + 1, (w + 1) & 1)
        _drain_extract(ut, idx_s, stag, u_flat, sem, w, w & 1)
        return 0

    lax.fori_loop(0, NWAVES - 1, wave, 0)
    _drain_extract(ut, idx_s, stag, u_flat, sem, NWAVES - 1, (NWAVES - 1) & 1)
    pltpu.sync_copy(u_flat, out_hbm.at[pl.ds(base * K, BPW * K)])


def _dot_body(it, iid, urows, out_hbm, idx_s, stag, u_flat, i_flat, out_v,
              sem):
    base = _worker_base()
    pltpu.sync_copy(iid.at[pl.ds(base, BPW)], idx_s)
    pltpu.sync_copy(urows.at[pl.ds(base * K, BPW * K)], u_flat)
    _fire(it, idx_s, stag, sem, 0, 0)

    def dot16(b0):
        flat0 = b0 * K + lax.iota(jnp.int32, L) * K
        acc = jnp.zeros((L,), jnp.float32)
        for k in range(K):
            u = plsc.load_gather(u_flat, [flat0 + k])
            v = plsc.load_gather(i_flat, [flat0 + k])
            acc = acc + u * v
        out_v[pl.ds(b0, L)] = acc

    def wave(w, _):
        _fire(it, idx_s, stag, sem, w + 1, (w + 1) & 1)
        b0 = _drain_extract(it, idx_s, stag, i_flat, sem, w, w & 1)
        dot16(b0)
        return 0

    lax.fori_loop(0, NWAVES - 1, wave, 0)
    b0 = _drain_extract(it, idx_s, stag, i_flat, sem, NWAVES - 1,
                        (NWAVES - 1) & 1)
    dot16(b0)
    pltpu.sync_copy(out_v, out_hbm.at[pl.ds(base, BPW)])


def kernel(user_id, item_id, user_table, item_table):
    ut = user_table.reshape(N // G, G, K)
    it = item_table.reshape(N // G, G, K)
    uid = user_id.astype(jnp.int32)
    iid = item_id.astype(jnp.int32)
    u_rows = pl.kernel(
        _gather_body,
        out_type=jax.ShapeDtypeStruct((B * K,), jnp.float32),
        mesh=plsc.VectorSubcoreMesh(**_MESH),
        scratch_types=[
            pltpu.VMEM((BPW,), jnp.int32),
            pltpu.VMEM((2 * W * G, K), jnp.float32),
            pltpu.VMEM((BPW * K,), jnp.float32),
            pltpu.SemaphoreType.DMA,
        ],
        compiler_params=_PARAMS,
    )(ut, uid)
    out = pl.kernel(
        _dot_body,
        out_type=jax.ShapeDtypeStruct((B,), jnp.float32),
        mesh=plsc.VectorSubcoreMesh(**_MESH),
        scratch_types=[
            pltpu.VMEM((BPW,), jnp.int32),
            pltpu.VMEM((2 * W * G, K), jnp.float32),
            pltpu.VMEM((BPW * K,), jnp.float32),
            pltpu.VMEM((BPW * K,), jnp.float32),
            pltpu.VMEM((BPW,), jnp.float32),
            pltpu.SemaphoreType.DMA,
        ],
        compiler_params=_PARAMS,
    )(it, iid, u_rows)
    return out.reshape(B, 1)


# speedup vs baseline: 1.0473x; 1.0473x over previous
import jax
import jax.numpy as jnp
from jax import lax
from jax.experimental import pallas as pl
from jax.experimental.pallas import tpu as pltpu
from jax.experimental.pallas import tpu_sc as plsc

B = 16384
K = 32
N = 1000000
G = 8
NC = 2
NS = 16
NW = NC * NS
BPW = B // NW
L = 16
W = 16


def _body(ut, it, uid, iid, out_hbm,
          idx_u_s, idx_i_s,
          stag_u, stag_i, u_flat, i_flat, out_v, sem):
    wid = lax.axis_index("s") * NC + lax.axis_index("c")
    base = wid * BPW

    pltpu.sync_copy(uid.at[pl.ds(base, BPW)], idx_u_s)
    pltpu.sync_copy(iid.at[pl.ds(base, BPW)], idx_i_s)

    def fire(w, half):
        b0 = pl.multiple_of(w * W, W)
        hb = pl.multiple_of(half * W * G, W * G)
        iv_u = idx_u_s[pl.ds(b0, W)]
        iv_i = idx_i_s[pl.ds(b0, W)]
        for t in range(W):
            gu = iv_u[t] >> 3
            gi = iv_i[t] >> 3
            pltpu.async_copy(ut.at[gu], stag_u.at[pl.ds(hb + t * G, G)], sem)
            pltpu.async_copy(it.at[gi], stag_i.at[pl.ds(hb + t * G, G)], sem)

    def drain_extract(w, half):
        b0 = pl.multiple_of(w * W, W)
        hb = pl.multiple_of(half * W * G, W * G)
        iv_u = idx_u_s[pl.ds(b0, W)]
        iv_i = idx_i_s[pl.ds(b0, W)]
        for t in range(W):
            pltpu.make_async_copy(
                ut.at[0], stag_u.at[pl.ds(hb + t * G, G)], sem).wait()
            pltpu.make_async_copy(
                it.at[0], stag_i.at[pl.ds(hb + t * G, G)], sem).wait()
        for t in range(W):
            b = b0 + t
            ru = hb + t * G + (iv_u[t] & 7)
            ri = hb + t * G + (iv_i[t] & 7)
            u_flat[pl.ds(b * K, L)] = stag_u[ru, pl.ds(0, L)]
            u_flat[pl.ds(b * K + L, L)] = stag_u[ru, pl.ds(L, L)]
            i_flat[pl.ds(b * K, L)] = stag_i[ri, pl.ds(0, L)]
            i_flat[pl.ds(b * K + L, L)] = stag_i[ri, pl.ds(L, L)]
        flat0 = b0 * K + lax.iota(jnp.int32, L) * K
        acc = jnp.zeros((L,), jnp.float32)
        for k in range(K):
            u = plsc.load_gather(u_flat, [flat0 + k])
            v = plsc.load_gather(i_flat, [flat0 + k])
            acc = acc + u * v
        out_v[pl.ds(b0, L)] = acc

    NWAVES = BPW // W
    fire(0, 0)

    def wave(w, _):
        fire(w + 1, (w + 1) & 1)
        drain_extract(w, w & 1)
        return 0

    lax.fori_loop(0, NWAVES - 1, wave, 0)
    drain_extract(NWAVES - 1, (NWAVES - 1) & 1)

    pltpu.sync_copy(out_v, out_hbm.at[pl.ds(base, BPW)])


def kernel(user_id, item_id, user_table, item_table):
    ut = user_table.reshape(N // G, G, K)
    it = item_table.reshape(N // G, G, K)
    uid = user_id.astype(jnp.int32)
    iid = item_id.astype(jnp.int32)
    mesh = plsc.VectorSubcoreMesh(core_axis_name="c", subcore_axis_name="s",
                                  num_cores=NC, num_subcores=NS)
    out = pl.kernel(
        _body,
        out_type=jax.ShapeDtypeStruct((B,), jnp.float32),
        mesh=mesh,
        scratch_types=[
            pltpu.VMEM((BPW,), jnp.int32),
            pltpu.VMEM((BPW,), jnp.int32),
            pltpu.VMEM((2 * W * G, K), jnp.float32),
            pltpu.VMEM((2 * W * G, K), jnp.float32),
            pltpu.VMEM((BPW * K,), jnp.float32),
            pltpu.VMEM((BPW * K,), jnp.float32),
            pltpu.VMEM((BPW,), jnp.float32),
            pltpu.SemaphoreType.DMA,
        ],
        compiler_params=pltpu.CompilerParams(needs_layout_passes=False,
                                             use_tc_tiling_on_sc=True),
    )(ut, it, uid, iid)
    return out.reshape(B, 1)


# speedup vs baseline: 1.0567x; 1.0090x over previous
import jax
import jax.numpy as jnp
from jax import lax
from jax.experimental import pallas as pl
from jax.experimental.pallas import tpu as pltpu
from jax.experimental.pallas import tpu_sc as plsc

B = 16384
K = 32
N = 1000000
G = 8
NC = 2
NS = 16
NW = NC * NS
BPW = B // NW
L = 16
W = 16


def _body(ut, it, uid, iid, out_hbm,
          idx_u_s, idx_i_s,
          stag_u, stag_i, out_v, sem):
    wid = lax.axis_index("s") * NC + lax.axis_index("c")
    base = wid * BPW

    pltpu.sync_copy(uid.at[pl.ds(base, BPW)], idx_u_s)
    pltpu.sync_copy(iid.at[pl.ds(base, BPW)], idx_i_s)

    def fire(w, half):
        b0 = pl.multiple_of(w * W, W)
        hb = pl.multiple_of(half * W * G, W * G)
        iv_u = idx_u_s[pl.ds(b0, W)]
        iv_i = idx_i_s[pl.ds(b0, W)]
        for t in range(W):
            gu = iv_u[t] >> 3
            gi = iv_i[t] >> 3
            pltpu.async_copy(ut.at[gu], stag_u.at[pl.ds(hb + t * G, G)], sem)
            pltpu.async_copy(it.at[gi], stag_i.at[pl.ds(hb + t * G, G)], sem)

    def drain_extract(w, half):
        b0 = pl.multiple_of(w * W, W)
        hb = pl.multiple_of(half * W * G, W * G)
        iv_u = idx_u_s[pl.ds(b0, W)]
        iv_i = idx_i_s[pl.ds(b0, W)]
        for t in range(W):
            pltpu.make_async_copy(
                ut.at[0], stag_u.at[pl.ds(hb + t * G, G)], sem).wait()
            pltpu.make_async_copy(
                it.at[0], stag_i.at[pl.ds(hb + t * G, G)], sem).wait()
        rvec_u = hb + lax.iota(jnp.int32, L) * G + (iv_u & 7)
        rvec_i = hb + lax.iota(jnp.int32, L) * G + (iv_i & 7)
        acc = jnp.zeros((L,), jnp.float32)
        for k in range(K):
            kv = jnp.full((L,), k, jnp.int32)
            u = plsc.load_gather(stag_u, [rvec_u, kv])
            v = plsc.load_gather(stag_i, [rvec_i, kv])
            acc = acc + u * v
        out_v[pl.ds(b0, L)] = acc

    NWAVES = BPW // W
    fire(0, 0)

    def wave(w, _):
        fire(w + 1, (w + 1) & 1)
        drain_extract(w, w & 1)
        return 0

    lax.fori_loop(0, NWAVES - 1, wave, 0)
    drain_extract(NWAVES - 1, (NWAVES - 1) & 1)

    pltpu.sync_copy(out_v, out_hbm.at[pl.ds(base, BPW)])


def kernel(user_id, item_id, user_table, item_table):
    ut = user_table.reshape(N // G, G, K)
    it = item_table.reshape(N // G, G, K)
    uid = user_id.astype(jnp.int32)
    iid = item_id.astype(jnp.int32)
    mesh = plsc.VectorSubcoreMesh(core_axis_name="c", subcore_axis_name="s",
                                  num_cores=NC, num_subcores=NS)
    out = pl.kernel(
        _body,
        out_type=jax.ShapeDtypeStruct((B,), jnp.float32),
        mesh=mesh,
        scratch_types=[
            pltpu.VMEM((BPW,), jnp.int32),
            pltpu.VMEM((BPW,), jnp.int32),
            pltpu.VMEM((2 * W * G, K), jnp.float32),
            pltpu.VMEM((2 * W * G, K), jnp.float32),
            pltpu.VMEM((BPW,), jnp.float32),
            pltpu.SemaphoreType.DMA,
        ],
        compiler_params=pltpu.CompilerParams(needs_layout_passes=False,
                                             use_tc_tiling_on_sc=True),
    )(ut, it, uid, iid)
    return out.reshape(B, 1)
